# baseline (speedup 1.0000x reference)
import jax
import jax.numpy as jnp
from jax import lax
from jax.experimental import pallas as pl
from jax.experimental.pallas import tpu as pltpu
from jax.experimental.pallas import tpu_sc as plsc
from jax._src.pallas import core as pallas_core
from jax._src.pallas import mpmd
from jax._src.pallas.mosaic import core as tpu_core

N_ROWS = 1_000_000
N_COLS = 64
N_OUT = 16_384

ALIGNED = 999_936
COLS_PER_SC = N_COLS // 2
SEG = N_OUT // 16
NSUB = 16


def _scs_body(xt, xtail, idxt, ot, col_a, col_b, idx_v, out_v,
              dsem_a, dsem_b, rdy_a, rdy_b, done_a, done_b,
              isem_a, isem_b, gsem, osem_a, osem_b):
    cid = lax.axis_index("c")
    j0 = cid * COLS_PER_SC

    def stage(col_ref, j, dsem, do_wait):
        copies = [
            (xt.at[j, pl.ds(0, ALIGNED)], col_ref.at[pl.ds(0, ALIGNED)]),
            (xtail.at[pl.ds(j * 128, 128)], col_ref.at[pl.ds(ALIGNED, 128)]),
        ]
        for s, d in copies:
            cp = pltpu.make_async_copy(s, d, dsem)
            cp.wait() if do_wait else cp.start()

    stage(col_a, j0, dsem_a, False)
    for k in range(COLS_PER_SC):
        buf, dsem = (col_a, dsem_a) if k % 2 == 0 else (col_b, dsem_b)
        rdy = rdy_a if k % 2 == 0 else rdy_b
        stage(buf, j0 + k, dsem, True)
        for t in range(NSUB):
            pltpu.semaphore_signal(rdy, 1, device_id={"s": t})
        if k + 1 < COLS_PER_SC:
            nbuf, ndsem = (col_b, dsem_b) if k % 2 == 0 else (col_a, dsem_a)
            if k >= 1:
                pltpu.semaphore_wait(done_b if k % 2 == 0 else done_a, NSUB)
            stage(nbuf, j0 + k + 1, ndsem, False)


def _tec_body(xt, xtail, idxt, ot, col_a, col_b, idx_v, out_v,
              dsem_a, dsem_b, rdy_a, rdy_b, done_a, done_b,
              isem_a, isem_b, gsem, osem_a, osem_b):
    cid = lax.axis_index("c")
    sid = lax.axis_index("s")
    j0 = cid * COLS_PER_SC

    def idx_slot(k):
        return idx_v.at[pl.ds((k % 2) * SEG, SEG)]

    def out_slot(k):
        return out_v.at[pl.ds((k % 2) * SEG, SEG)]

    def idx_copy(k, do_wait):
        cp = pltpu.make_async_copy(idxt.at[j0 + k, pl.ds(sid * SEG, SEG)],
                                   idx_slot(k), isem_a if k % 2 == 0 else isem_b)
        cp.wait() if do_wait else cp.start()

    def out_copy(k, do_wait):
        cp = pltpu.make_async_copy(out_slot(k), ot.at[j0 + k, pl.ds(sid * SEG, SEG)],
                                   osem_a if k % 2 == 0 else osem_b)
        cp.wait() if do_wait else cp.start()

    idx_copy(0, False)
    for k in range(COLS_PER_SC):
        buf = col_a if k % 2 == 0 else col_b
        rdy = rdy_a if k % 2 == 0 else rdy_b
        done = done_a if k % 2 == 0 else done_b
        if k + 1 < COLS_PER_SC:
            idx_copy(k + 1, False)
        idx_copy(k, True)
        if k >= 2:
            out_copy(k - 2, True)
        pl.semaphore_wait(rdy, 1)

        pltpu.make_async_copy(buf.at[idx_slot(k)], out_slot(k), gsem).start()
        pltpu.make_async_copy(buf.at[idx_slot(k)], out_slot(k), gsem).wait()
        pltpu.semaphore_signal(done, 1)
        out_copy(k, False)

    out_copy(COLS_PER_SC - 2, True)
    out_copy(COLS_PER_SC - 1, True)


@jax.jit
def _gather_sc(xt, xtail, idxt):
    smesh = plsc.ScalarSubcoreMesh(axis_name="c", num_cores=2)
    vmesh = plsc.VectorSubcoreMesh(core_axis_name="c", subcore_axis_name="s")
    sem = tpu_core.MemorySpace.SEMAPHORE
    dma_aval = pltpu.SemaphoreType.DMA(()).inner_aval
    reg_aval = pltpu.SemaphoreType.REGULAR(()).inner_aval

    def scs_sem(aval):
        return pallas_core.MemoryRef(aval, pallas_core.CoreMemorySpace(sem, smesh))

    def tec_sem(aval):
        return pallas_core.MemoryRef(aval, pallas_core.CoreMemorySpace(sem, vmesh))

    tec_vmem = pallas_core.CoreMemorySpace(tpu_core.MemorySpace.VMEM, vmesh)

    return mpmd.mpmd_map(
        [(smesh, _scs_body), (vmesh, _tec_body)],
        out_types=jax.ShapeDtypeStruct((N_COLS, N_OUT), jnp.float32),
        scratch_types=[
            pltpu.VMEM_SHARED((ALIGNED + 128,), jnp.float32),
            pltpu.VMEM_SHARED((ALIGNED + 128,), jnp.float32),
            tec_vmem((2 * SEG,), jnp.int32),
            tec_vmem((2 * SEG,), jnp.float32),
            scs_sem(dma_aval),
            scs_sem(dma_aval),
            tec_sem(reg_aval),
            tec_sem(reg_aval),
            scs_sem(reg_aval),
            scs_sem(reg_aval),
            tec_sem(dma_aval),
            tec_sem(dma_aval),
            tec_sem(dma_aval),
            tec_sem(dma_aval),
            tec_sem(dma_aval),
        ],
    )(xt, xtail, idxt)


def kernel(x, dim, index, sparse_grad, out):
    xtail = jnp.pad(x[ALIGNED:, :], ((0, 128 - (N_ROWS - ALIGNED)), (0, 0)))
    res_t = _gather_sc(x.T, xtail.T.reshape(-1), index.astype(jnp.int32).T)
    return res_t.T
